# baseline (speedup 1.0000x reference)
import jax
import jax.numpy as jnp
from jax import lax
from jax.experimental import pallas as pl
from jax.experimental.pallas import tpu as pltpu
from jax.experimental.pallas import tpu_sc as plsc

NC = 2
NS = 16
NW = NC * NS
PB = 128
DEG_W = 8


def _sc_degree(col3d, nb, acc_rows, n):
    mesh = plsc.VectorSubcoreMesh(core_axis_name="c", subcore_axis_name="s")
    nzb = acc_rows // PB

    def body(col_hbm, out_hbm, cbuf, ones, acc):
        cid = lax.axis_index("c")
        sid = lax.axis_index("s")
        wid = sid * NC + cid

        def fill(i, _):
            ones[i, :] = jnp.zeros((DEG_W,), jnp.float32)
            return 0
        lax.fori_loop(0, PB, fill, 0)

        def zacc(k, _):
            b = k * NS + sid

            @pl.when(b < nzb)
            def _():
                pltpu.sync_copy(ones, acc.at[pl.ds(b * PB, PB)])
            return 0
        lax.fori_loop(0, -(-nzb // NS), zacc, 0)

        def fill1(i, _):
            ones[i, :] = jnp.ones((DEG_W,), jnp.float32)
            return 0
        lax.fori_loop(0, PB, fill1, 0)
        plsc.subcore_barrier()

        pltpu.sync_copy(col_hbm.at[wid], cbuf)

        def step(j, _):
            pltpu.sync_copy(ones, acc.at[cbuf.at[j]], add=True)
            return 0
        lax.fori_loop(0, nb, step, 0)
        plsc.subcore_barrier()

        @pl.when(sid == 0)
        def _():
            pltpu.sync_copy(acc.at[pl.ds(0, n)], out_hbm.at[cid])

    return pl.kernel(
        body,
        out_type=jax.ShapeDtypeStruct((NC, n, DEG_W), jnp.float32),
        mesh=mesh,
        scratch_types=[
            pltpu.VMEM((nb, PB), jnp.int32),
            pltpu.VMEM((PB, DEG_W), jnp.float32),
            pltpu.VMEM_SHARED((acc_rows, DEG_W), jnp.float32),
        ],
    )(col3d)


def _sc_scatter(hs, row3d, col3d, nb, acc_rows, n, d):
    mesh = plsc.VectorSubcoreMesh(core_axis_name="c", subcore_axis_name="s")
    nzb = acc_rows // PB

    def body(hs_hbm, row_hbm, col_hbm, out_hbm, rbuf, cbuf, rows, acc, sem):
        cid = lax.axis_index("c")
        sid = lax.axis_index("s")
        wid = sid * NC + cid

        def zrow(i, _):
            def zcol(j, _):
                rows[i, pl.ds(j * 16, 16)] = jnp.zeros((16,), jnp.float32)
                return 0
            return lax.fori_loop(0, d // 16, zcol, 0)
        lax.fori_loop(0, PB, zrow, 0)

        def zacc(k, _):
            b = k * NS + sid

            @pl.when(b < nzb)
            def _():
                pltpu.sync_copy(rows, acc.at[pl.ds(b * PB, PB)])
            return 0
        lax.fori_loop(0, -(-nzb // NS), zacc, 0)
        plsc.subcore_barrier()

        pltpu.sync_copy(row_hbm.at[wid], rbuf)
        pltpu.sync_copy(col_hbm.at[wid], cbuf)

        def step(j, _):
            pltpu.async_copy(hs_hbm.at[rbuf.at[j]], rows, sem).wait()
            pltpu.sync_copy(rows, acc.at[cbuf.at[j]], add=True)
            return 0
        lax.fori_loop(0, nb, step, 0)
        plsc.subcore_barrier()

        @pl.when(sid == 0)
        def _():
            pltpu.sync_copy(acc.at[pl.ds(0, n)], out_hbm.at[cid])

    return pl.kernel(
        body,
        out_type=jax.ShapeDtypeStruct((NC, n, d), jnp.float32),
        mesh=mesh,
        scratch_types=[
            pltpu.VMEM((nb, PB), jnp.int32),
            pltpu.VMEM((nb, PB), jnp.int32),
            pltpu.VMEM((PB, d), jnp.float32),
            pltpu.VMEM_SHARED((acc_rows, d), jnp.float32),
            pltpu.SemaphoreType.DMA,
        ],
    )(hs, row3d, col3d)


def _dinv_of(dp_ref):
    dtot = dp_ref[0, :, 0:1] + dp_ref[1, :, 0:1] + 1.0
    return lax.rsqrt(dtot)


def _tc_first(x, w1, degp, blk):
    n, din = x.shape
    dh = w1.shape[1]

    def body(x_ref, w_ref, dp_ref, o_ref):
        dinv = _dinv_of(dp_ref)
        h = jnp.dot(x_ref[...], w_ref[...], preferred_element_type=jnp.float32)
        o_ref[...] = h * dinv

    return pl.pallas_call(
        body,
        grid=(n // blk,),
        in_specs=[
            pl.BlockSpec((blk, din), lambda i: (i, 0)),
            pl.BlockSpec((din, dh), lambda i: (0, 0)),
            pl.BlockSpec((2, blk, DEG_W), lambda i: (0, i, 0)),
        ],
        out_specs=pl.BlockSpec((blk, dh), lambda i: (i, 0)),
        out_shape=jax.ShapeDtypeStruct((n, dh), jnp.float32),
    )(x, w1, degp)


def _tc_mid(p1, hs1, degp, b1, w2, blk):
    n, dh = hs1.shape
    dout = w2.shape[1]

    def body(p_ref, hs_ref, dp_ref, b_ref, w_ref, o_ref):
        dinv = _dinv_of(dp_ref)
        s = (p_ref[0] + p_ref[1] + hs_ref[...]) * dinv
        z = jnp.maximum(s + b_ref[...], 0.0)
        h = jnp.dot(z, w_ref[...], preferred_element_type=jnp.float32)
        o_ref[...] = h * dinv

    return pl.pallas_call(
        body,
        grid=(n // blk,),
        in_specs=[
            pl.BlockSpec((2, blk, dh), lambda i: (0, i, 0)),
            pl.BlockSpec((blk, dh), lambda i: (i, 0)),
            pl.BlockSpec((2, blk, DEG_W), lambda i: (0, i, 0)),
            pl.BlockSpec((1, dh), lambda i: (0, 0)),
            pl.BlockSpec((dh, dout), lambda i: (0, 0)),
        ],
        out_specs=pl.BlockSpec((blk, dout), lambda i: (i, 0)),
        out_shape=jax.ShapeDtypeStruct((n, dout), jnp.float32),
    )(p1, hs1, degp, b1, w2)


def _tc_last(p2, hs2, degp, b2, blk):
    n, dout = hs2.shape

    def body(p_ref, hs_ref, dp_ref, b_ref, o_ref):
        dinv = _dinv_of(dp_ref)
        o_ref[...] = (p_ref[0] + p_ref[1] + hs_ref[...]) * dinv + b_ref[...]

    return pl.pallas_call(
        body,
        grid=(n // blk,),
        in_specs=[
            pl.BlockSpec((2, blk, dout), lambda i: (0, i, 0)),
            pl.BlockSpec((blk, dout), lambda i: (i, 0)),
            pl.BlockSpec((2, blk, DEG_W), lambda i: (0, i, 0)),
            pl.BlockSpec((1, dout), lambda i: (0, 0)),
        ],
        out_specs=pl.BlockSpec((blk, dout), lambda i: (i, 0)),
        out_shape=jax.ShapeDtypeStruct((n, dout), jnp.float32),
    )(p2, hs2, degp, b2)


def kernel(x, edge_index, W1, b1, W2, b2):
    n, din = x.shape
    dh = W1.shape[1]
    dout = W2.shape[1]
    e = edge_index.shape[1]

    row = edge_index[0]
    col = edge_index[1]
    nb = -(-e // (NW * PB))
    nb1 = nb
    nb2 = nb1 + 1

    def padded(idx, fill, nbk):
        padn = NW * PB * nbk - e
        return jnp.concatenate(
            [idx, jnp.full((padn,), fill, jnp.int32)]).reshape(NW, nbk, PB)

    row_p1 = padded(row, 0, nb1)
    col_p1 = padded(col, n, nb1)
    row_p2 = padded(row, 0, nb2)
    col_p2 = padded(col, n, nb2)

    acc_rows = -(-(n + 1) // PB) * PB
    blk = 2000

    degp = _sc_degree(col_p1, nb1, acc_rows, n)
    hs1 = _tc_first(x, W1, degp, blk)
    p1 = _sc_scatter(hs1, row_p1, col_p1, nb1, acc_rows, n, dh)
    hs2 = _tc_mid(p1, hs1, degp, b1.reshape(1, dh), W2, blk)
    p2 = _sc_scatter(hs2, row_p2, col_p2, nb2, acc_rows, n, dh)
    out = _tc_last(p2, hs2, degp, b2.reshape(1, dout), blk)
    return out


# speedup vs baseline: 1.3144x; 1.3144x over previous
import jax
import jax.numpy as jnp
from jax import lax
from jax.experimental import pallas as pl
from jax.experimental.pallas import tpu as pltpu
from jax.experimental.pallas import tpu_sc as plsc

NC = 2
NS = 16
NW = NC * NS
PB = 128
DEG_W = 16


def _sc_degree(col3d, nb, acc_rows, n):
    mesh = plsc.VectorSubcoreMesh(core_axis_name="c", subcore_axis_name="s")
    nz = acc_rows // (NS * PB)

    def body(col_hbm, out_hbm, cbuf, ones, acc):
        cid = lax.axis_index("c")
        sid = lax.axis_index("s")
        wid = sid * NC + cid

        def fill(i, _):
            ones[i, :] = jnp.zeros((DEG_W,), jnp.float32)
            return 0
        lax.fori_loop(0, PB, fill, 0)

        def zacc(k, _):
            pltpu.sync_copy(ones, acc.at[pl.ds((sid * nz + k) * PB, PB)])
            return 0
        lax.fori_loop(0, nz, zacc, 0)

        def fill1(i, _):
            ones[i, :] = jnp.ones((DEG_W,), jnp.float32)
            return 0
        lax.fori_loop(0, PB, fill1, 0)
        plsc.subcore_barrier()

        pltpu.sync_copy(col_hbm.at[wid], cbuf)

        def step(j, _):
            pltpu.sync_copy(ones, acc.at[cbuf.at[j]], add=True)
            return 0
        lax.fori_loop(0, nb, step, 0)
        plsc.subcore_barrier()

        @pl.when(sid == 0)
        def _():
            pltpu.sync_copy(acc.at[pl.ds(0, n)], out_hbm.at[cid])

    return pl.kernel(
        body,
        out_type=jax.ShapeDtypeStruct((NC, n, DEG_W), jnp.float32),
        mesh=mesh,
        scratch_types=[
            pltpu.VMEM((nb, PB), jnp.int32),
            pltpu.VMEM((PB, DEG_W), jnp.float32),
            pltpu.VMEM_SHARED((acc_rows, DEG_W), jnp.float32),
        ],
    )(col3d)


def _sc_scatter(hs, row3d, col3d, nb, acc_rows, n, d):
    mesh = plsc.VectorSubcoreMesh(core_axis_name="c", subcore_axis_name="s")
    nz = acc_rows // (NS * PB)

    def body(hs_hbm, row_hbm, col_hbm, out_hbm, rbuf, cbuf, rows, acc, sem):
        cid = lax.axis_index("c")
        sid = lax.axis_index("s")
        wid = sid * NC + cid

        def zrow(i, _):
            def zcol(j, _):
                rows[i, pl.ds(j * 16, 16)] = jnp.zeros((16,), jnp.float32)
                return 0
            return lax.fori_loop(0, d // 16, zcol, 0)
        lax.fori_loop(0, PB, zrow, 0)

        def zacc(k, _):
            pltpu.sync_copy(rows, acc.at[pl.ds((sid * nz + k) * PB, PB)])
            return 0
        lax.fori_loop(0, nz, zacc, 0)
        plsc.subcore_barrier()

        pltpu.sync_copy(row_hbm.at[wid], rbuf)
        pltpu.sync_copy(col_hbm.at[wid], cbuf)

        def step(j, _):
            pltpu.async_copy(hs_hbm.at[rbuf.at[j]], rows, sem).wait()
            pltpu.sync_copy(rows, acc.at[cbuf.at[j]], add=True)
            return 0
        lax.fori_loop(0, nb, step, 0)
        plsc.subcore_barrier()

        @pl.when(sid == 0)
        def _():
            pltpu.sync_copy(acc.at[pl.ds(0, n)], out_hbm.at[cid])

    return pl.kernel(
        body,
        out_type=jax.ShapeDtypeStruct((NC, n, d), jnp.float32),
        mesh=mesh,
        scratch_types=[
            pltpu.VMEM((nb, PB), jnp.int32),
            pltpu.VMEM((nb, PB), jnp.int32),
            pltpu.VMEM((PB, d), jnp.float32),
            pltpu.VMEM_SHARED((acc_rows, d), jnp.float32),
            pltpu.SemaphoreType.DMA,
        ],
    )(hs, row3d, col3d)


def _dinv_of(dp_ref):
    dtot = dp_ref[0, :, 0:1] + dp_ref[1, :, 0:1] + 1.0
    return lax.rsqrt(dtot)


def _tc_first(x, w1, degp, blk):
    n, din = x.shape
    dh = w1.shape[1]

    def body(x_ref, w_ref, dp_ref, o_ref):
        dinv = _dinv_of(dp_ref)
        h = jnp.dot(x_ref[...], w_ref[...], preferred_element_type=jnp.float32)
        o_ref[...] = h * dinv

    return pl.pallas_call(
        body,
        grid=(n // blk,),
        in_specs=[
            pl.BlockSpec((blk, din), lambda i: (i, 0)),
            pl.BlockSpec((din, dh), lambda i: (0, 0)),
            pl.BlockSpec((2, blk, DEG_W), lambda i: (0, i, 0)),
        ],
        out_specs=pl.BlockSpec((blk, dh), lambda i: (i, 0)),
        out_shape=jax.ShapeDtypeStruct((n, dh), jnp.float32),
    )(x, w1, degp)


def _tc_mid(p1, hs1, degp, b1, w2, blk):
    n, dh = hs1.shape
    dout = w2.shape[1]

    def body(p_ref, hs_ref, dp_ref, b_ref, w_ref, o_ref):
        dinv = _dinv_of(dp_ref)
        s = (p_ref[0] + p_ref[1] + hs_ref[...]) * dinv
        z = jnp.maximum(s + b_ref[...], 0.0)
        h = jnp.dot(z, w_ref[...], preferred_element_type=jnp.float32)
        o_ref[...] = h * dinv

    return pl.pallas_call(
        body,
        grid=(n // blk,),
        in_specs=[
            pl.BlockSpec((2, blk, dh), lambda i: (0, i, 0)),
            pl.BlockSpec((blk, dh), lambda i: (i, 0)),
            pl.BlockSpec((2, blk, DEG_W), lambda i: (0, i, 0)),
            pl.BlockSpec((1, dh), lambda i: (0, 0)),
            pl.BlockSpec((dh, dout), lambda i: (0, 0)),
        ],
        out_specs=pl.BlockSpec((blk, dout), lambda i: (i, 0)),
        out_shape=jax.ShapeDtypeStruct((n, dout), jnp.float32),
    )(p1, hs1, degp, b1, w2)


def _tc_last(p2, hs2, degp, b2, blk):
    n, dout = hs2.shape

    def body(p_ref, hs_ref, dp_ref, b_ref, o_ref):
        dinv = _dinv_of(dp_ref)
        o_ref[...] = (p_ref[0] + p_ref[1] + hs_ref[...]) * dinv + b_ref[...]

    return pl.pallas_call(
        body,
        grid=(n // blk,),
        in_specs=[
            pl.BlockSpec((2, blk, dout), lambda i: (0, i, 0)),
            pl.BlockSpec((blk, dout), lambda i: (i, 0)),
            pl.BlockSpec((2, blk, DEG_W), lambda i: (0, i, 0)),
            pl.BlockSpec((1, dout), lambda i: (0, 0)),
        ],
        out_specs=pl.BlockSpec((blk, dout), lambda i: (i, 0)),
        out_shape=jax.ShapeDtypeStruct((n, dout), jnp.float32),
    )(p2, hs2, degp, b2)


def kernel(x, edge_index, W1, b1, W2, b2):
    n, din = x.shape
    dh = W1.shape[1]
    dout = W2.shape[1]
    e = edge_index.shape[1]

    row = edge_index[0]
    col = edge_index[1]
    nb = -(-e // (NW * PB))
    e_pad = NW * PB * nb
    pad = e_pad - e
    row_p = jnp.concatenate([row, jnp.zeros((pad,), jnp.int32)]).reshape(NW, nb, PB)
    col_p = jnp.concatenate([col, jnp.full((pad,), n, jnp.int32)]).reshape(NW, nb, PB)

    tile_rows = NS * PB
    acc_rows = -(-(n + 1) // tile_rows) * tile_rows
    blk = 2000

    degp = _sc_degree(col_p, nb, acc_rows, n)
    hs1 = _tc_first(x, W1, degp, blk)
    p1 = _sc_scatter(hs1, row_p, col_p, nb, acc_rows, n, dh)
    hs2 = _tc_mid(p1, hs1, degp, b1.reshape(1, dh), W2, blk)
    p2 = _sc_scatter(hs2, row_p, col_p, nb, acc_rows, n, dh)
    out = _tc_last(p2, hs2, degp, b2.reshape(1, dout), blk)
    return out
